# initial kernel scaffold (unmeasured)
import jax
import jax.numpy as jnp
from jax import lax
from jax.experimental import pallas as pl
from jax.experimental.pallas import tpu as pltpu


def kernel(
    x,
):
    def body(*refs):
        pass

    out_shape = jax.ShapeDtypeStruct(..., jnp.float32)
    return pl.pallas_call(body, out_shape=out_shape)(...)


# baseline (device time: 109760 ns/iter reference)
import functools

import jax
import jax.numpy as jnp
from jax import lax
from jax.experimental import pallas as pl
from jax.experimental.pallas import tpu as pltpu

N_Z = 4


def kernel(x):
    m_per, n = x.shape
    n_per = n // N_Z
    m_total = m_per * N_Z

    def body(x_ref, out_ref, sbf_ref, rbf_ref, send_sems, recv_sems):
        my_x = lax.axis_index("x")
        my_y = lax.axis_index("y")
        my_z = lax.axis_index("z")

        barrier_sem = pltpu.get_barrier_semaphore()
        for d in range(1, N_Z):
            pl.semaphore_signal(
                barrier_sem, inc=1,
                device_id=(my_x, my_y, (my_z + d) % N_Z),
                device_id_type=pl.DeviceIdType.MESH,
            )
        pl.semaphore_wait(barrier_sem, N_Z - 1)

        rdmas = []
        for d in range(1, N_Z):
            tz = (my_z + d) % N_Z
            sbf_ref[d - 1, :, :] = x_ref[:, pl.ds(tz * n_per, n_per)].astype(
                jnp.bfloat16
            )
            rdma = pltpu.make_async_remote_copy(
                src_ref=sbf_ref.at[d - 1],
                dst_ref=rbf_ref.at[d - 1],
                send_sem=send_sems.at[d - 1],
                recv_sem=recv_sems.at[d - 1],
                device_id=(my_x, my_y, tz),
                device_id_type=pl.DeviceIdType.MESH,
            )
            rdma.start()
            rdmas.append(rdma)

        out_ref[pl.ds(my_z * m_per, m_per), :] = x_ref[:, pl.ds(my_z * n_per, n_per)]

        for j, rdma in enumerate(rdmas):
            rdma.wait_recv()
            origin = (my_z - (j + 1)) % N_Z
            out_ref[pl.ds(origin * m_per, m_per), :] = rbf_ref[j, :, :].astype(
                jnp.float32
            )
        for rdma in rdmas:
            rdma.wait_send()

        @functools.partial(
            pl.run_scoped, second_barrier=pltpu.SemaphoreType.REGULAR
        )
        def _(second_barrier):
            for d in range(1, N_Z):
                pl.semaphore_signal(
                    second_barrier, inc=1,
                    device_id=(my_x, my_y, (my_z + d) % N_Z),
                    device_id_type=pl.DeviceIdType.MESH,
                )
            pl.semaphore_wait(second_barrier, N_Z - 1)

    return pl.pallas_call(
        body,
        out_shape=jax.ShapeDtypeStruct((m_total, n_per), jnp.float32),
        in_specs=[pl.BlockSpec(memory_space=pltpu.VMEM)],
        out_specs=pl.BlockSpec(memory_space=pltpu.VMEM),
        scratch_shapes=[
            pltpu.VMEM((N_Z - 1, m_per, n_per), jnp.bfloat16),
            pltpu.VMEM((N_Z - 1, m_per, n_per), jnp.bfloat16),
            pltpu.SemaphoreType.DMA((N_Z - 1,)),
            pltpu.SemaphoreType.DMA((N_Z - 1,)),
        ],
        compiler_params=pltpu.CompilerParams(collective_id=0),
    )(x)


# device time: 106955 ns/iter; 1.0262x vs baseline; 1.0262x over previous
import functools

import jax
import jax.numpy as jnp
from jax import lax
from jax.experimental import pallas as pl
from jax.experimental.pallas import tpu as pltpu

N_Z = 4


def kernel(x):
    m_per, n = x.shape
    n_per = n // N_Z
    m_total = m_per * N_Z

    def body(x_ref, out_ref, xs_ref, sbf_ref, rbf_ref,
             in_sems, out_sems, own_sem, send_sems, recv_sems):
        my_x = lax.axis_index("x")
        my_y = lax.axis_index("y")
        my_z = lax.axis_index("z")

        barrier_sem = pltpu.get_barrier_semaphore()
        for d in range(1, N_Z):
            pl.semaphore_signal(
                barrier_sem, inc=1,
                device_id=(my_x, my_y, (my_z + d) % N_Z),
                device_id_type=pl.DeviceIdType.MESH,
            )
        pl.semaphore_wait(barrier_sem, N_Z - 1)

        in_dmas = []
        for d in range(1, N_Z):
            tz = (my_z + d) % N_Z
            cp = pltpu.make_async_copy(
                x_ref.at[:, pl.ds(tz * n_per, n_per)],
                xs_ref.at[d - 1],
                in_sems.at[d - 1],
            )
            cp.start()
            in_dmas.append(cp)
        own = pltpu.make_async_copy(
            x_ref.at[:, pl.ds(my_z * n_per, n_per)],
            out_ref.at[pl.ds(my_z * m_per, m_per), :],
            own_sem,
        )
        own.start()

        rdmas = []
        for d in range(1, N_Z):
            tz = (my_z + d) % N_Z
            in_dmas[d - 1].wait()
            sbf_ref[d - 1, :, :] = xs_ref[d - 1, :, :].astype(jnp.bfloat16)
            rdma = pltpu.make_async_remote_copy(
                src_ref=sbf_ref.at[d - 1],
                dst_ref=rbf_ref.at[d - 1],
                send_sem=send_sems.at[d - 1],
                recv_sem=recv_sems.at[d - 1],
                device_id=(my_x, my_y, tz),
                device_id_type=pl.DeviceIdType.MESH,
            )
            rdma.start()
            rdmas.append(rdma)

        out_dmas = []
        for j, rdma in enumerate(rdmas):
            rdma.wait_recv()
            xs_ref[j, :, :] = rbf_ref[j, :, :].astype(jnp.float32)
            origin = (my_z - (j + 1)) % N_Z
            cp = pltpu.make_async_copy(
                xs_ref.at[j],
                out_ref.at[pl.ds(origin * m_per, m_per), :],
                out_sems.at[j],
            )
            cp.start()
            out_dmas.append(cp)

        own.wait()
        for cp in out_dmas:
            cp.wait()
        for rdma in rdmas:
            rdma.wait_send()

        @functools.partial(
            pl.run_scoped, second_barrier=pltpu.SemaphoreType.REGULAR
        )
        def _(second_barrier):
            for d in range(1, N_Z):
                pl.semaphore_signal(
                    second_barrier, inc=1,
                    device_id=(my_x, my_y, (my_z + d) % N_Z),
                    device_id_type=pl.DeviceIdType.MESH,
                )
            pl.semaphore_wait(second_barrier, N_Z - 1)

    return pl.pallas_call(
        body,
        out_shape=jax.ShapeDtypeStruct((m_total, n_per), jnp.float32),
        in_specs=[pl.BlockSpec(memory_space=pl.ANY)],
        out_specs=pl.BlockSpec(memory_space=pl.ANY),
        scratch_shapes=[
            pltpu.VMEM((N_Z - 1, m_per, n_per), jnp.float32),
            pltpu.VMEM((N_Z - 1, m_per, n_per), jnp.bfloat16),
            pltpu.VMEM((N_Z - 1, m_per, n_per), jnp.bfloat16),
            pltpu.SemaphoreType.DMA((N_Z - 1,)),
            pltpu.SemaphoreType.DMA((N_Z - 1,)),
            pltpu.SemaphoreType.DMA,
            pltpu.SemaphoreType.DMA((N_Z - 1,)),
            pltpu.SemaphoreType.DMA((N_Z - 1,)),
        ],
        compiler_params=pltpu.CompilerParams(collective_id=0),
    )(x)
